# baseline (speedup 1.0000x reference)
import functools

import jax
import jax.numpy as jnp
from jax import lax
from jax.experimental import pallas as pl
from jax.experimental.pallas import tpu as pltpu
from jax.experimental.pallas import tpu_sc as plsc

_EPS = 1e-6
_NW = 32
_CHUNK = 32
_FEAT = 512
_LANES = 128


def _store_scalar(ref, i, val, lane):
    idx = jnp.broadcast_to(i, (16,)).astype(jnp.int32)
    v = jnp.broadcast_to(val, (16,))
    plsc.store_scatter(ref, [idx], v, mask=lane == 0)


def _sqrt16(a):
    a = jnp.maximum(a, 1e-12)
    y = plsc.bitcast(jnp.int32(0x5F3759DF) - (plsc.bitcast(a, jnp.int32) >> 1),
                     jnp.float32)
    for _ in range(3):
        y = y * (1.5 - 0.5 * a * y * y)
    return a * y


def _argmax_kernel(preds_ref, labels_ref, adv_ref):
    labels = labels_ref[...]
    b, c = preds_ref.shape
    col = lax.broadcasted_iota(jnp.int32, (b, _LANES), 1)
    cur_max = jnp.full((b, _LANES), -jnp.inf, jnp.float32)
    cur_idx = jnp.zeros((b, _LANES), jnp.int32)
    offs = [j * _LANES for j in range(c // _LANES)]
    if c % _LANES:
        offs.append(c - _LANES)
    for off in offs:
        v = preds_ref[:, off:off + _LANES]
        cc = col + off
        v = jnp.where(cc == labels, -jnp.inf, v)
        upd = v > cur_max
        cur_idx = jnp.where(upd, cc, cur_idx)
        cur_max = jnp.maximum(v, cur_max)
    gmax = jnp.max(cur_max, axis=1, keepdims=True)
    cand = jnp.where(cur_max == gmax, cur_idx, c)
    adv_ref[...] = jnp.min(cand, axis=1, keepdims=True)


def _sc_body(x_hbm, lab_hbm, adv_hbm, cent_hbm, out_hbm,
             x_v, pos_v, neg_v, lab_v, adv_v, d2p_v, d2n_v, res_v,
             sem_x0, sem_x1, sem_g0, sem_g1, *, rows_per_worker):
    cid = lax.axis_index("c")
    sid = lax.axis_index("s")
    wid = sid * 2 + cid
    lane = lax.broadcasted_iota(jnp.int32, (16,), 0)
    nchunks = rows_per_worker // _CHUNK
    sems_x = (sem_x0, sem_x1)
    sems_g = (sem_g0, sem_g1)

    def start(g):
        b = g % 2
        base = wid * rows_per_worker + g * _CHUNK
        pltpu.sync_copy(lab_hbm.at[pl.ds(base, _CHUNK)], lab_v.at[b])
        pltpu.sync_copy(adv_hbm.at[pl.ds(base, _CHUNK)], adv_v.at[b])
        cp_x = pltpu.async_copy(
            x_hbm.at[pl.ds(base, _CHUNK), :], x_v.at[b], sems_x[b])
        gp = pltpu.async_copy(cent_hbm.at[lab_v.at[b]], pos_v.at[b],
                              sems_g[b])
        gn = pltpu.async_copy(cent_hbm.at[adv_v.at[b]], neg_v.at[b],
                              sems_g[b])
        return cp_x, gp, gn

    def finish(g, cp_x, gp, gn, acc):
        b = g % 2
        cp_x.wait()
        gp.wait()
        gn.wait()

        def d2_row(r, _):
            accp = jnp.zeros((16,), jnp.float32)
            accn = jnp.zeros((16,), jnp.float32)
            for j in range(_FEAT // 16):
                xa = x_v[b, r, pl.ds(j * 16, 16)]
                tp = xa - pos_v[b, r, pl.ds(j * 16, 16)] + _EPS
                tn = xa - neg_v[b, r, pl.ds(j * 16, 16)] + _EPS
                accp = accp + tp * tp
                accn = accn + tn * tn
            _store_scalar(d2p_v, r, plsc.cumsum(accp)[15], lane)
            _store_scalar(d2n_v, r, plsc.cumsum(accn)[15], lane)
            return 0

        lax.fori_loop(0, _CHUNK, d2_row, 0)
        for h in range(_CHUNK // 16):
            d_ap = _sqrt16(d2p_v[pl.ds(h * 16, 16)])
            d_an = _sqrt16(d2n_v[pl.ds(h * 16, 16)])
            acc = acc + jnp.maximum(d_ap - d_an + 1.0, 0.0)
        return acc

    acc = jnp.zeros((16,), jnp.float32)
    hands = [start(0)]
    for g in range(nchunks):
        if g + 1 < nchunks:
            hands.append(start(g + 1))
        acc = finish(g, *hands[g], acc)

    s = plsc.cumsum(acc)[15]
    res_v[...] = jnp.broadcast_to(s, (16,))
    pltpu.sync_copy(res_v, out_hbm.at[wid])


def kernel(x, preds, labels, centers):
    batch, feat = x.shape
    num_classes = centers.shape[0]
    rows_per_worker = batch // _NW
    lab32 = labels.astype(jnp.int32)

    blk = 1024
    adv = pl.pallas_call(
        _argmax_kernel,
        grid=(batch // blk,),
        in_specs=[
            pl.BlockSpec((blk, num_classes), lambda i: (i, 0)),
            pl.BlockSpec((blk, 1), lambda i: (i, 0)),
        ],
        out_specs=pl.BlockSpec((blk, 1), lambda i: (i, 0)),
        out_shape=jax.ShapeDtypeStruct((batch, 1), jnp.int32),
    )(preds, lab32.reshape(batch, 1))

    partials = pl.kernel(
        functools.partial(_sc_body, rows_per_worker=rows_per_worker),
        out_type=jax.ShapeDtypeStruct((_NW, 16), jnp.float32),
        mesh=plsc.VectorSubcoreMesh(core_axis_name="c", subcore_axis_name="s"),
        compiler_params=pltpu.CompilerParams(needs_layout_passes=False),
        scratch_types=[
            pltpu.VMEM((2, _CHUNK, _FEAT), jnp.float32),
            pltpu.VMEM((2, _CHUNK, _FEAT), jnp.float32),
            pltpu.VMEM((2, _CHUNK, _FEAT), jnp.float32),
            pltpu.VMEM((2, _CHUNK), jnp.int32),
            pltpu.VMEM((2, _CHUNK), jnp.int32),
            pltpu.VMEM((_CHUNK,), jnp.float32),
            pltpu.VMEM((_CHUNK,), jnp.float32),
            pltpu.VMEM((16,), jnp.float32),
            pltpu.SemaphoreType.DMA,
            pltpu.SemaphoreType.DMA,
            pltpu.SemaphoreType.DMA,
            pltpu.SemaphoreType.DMA,
        ],
    )(x, lab32, adv.reshape(batch), centers)

    return jnp.sum(partials[:, 0]) * (1.0 / batch)
